# baseline (speedup 1.0000x reference)
import functools

import jax
import jax.numpy as jnp
from jax import lax
from jax.experimental import pallas as pl
from jax.experimental.pallas import tpu as pltpu
from jax.experimental.pallas import tpu_sc as plsc

NC = 2
NS = 16
NW = NC * NS
L = 16

CHUNK = 96


def _dense_body(e_ref, q_ref, k_ref, v_ref, res_ref, att_ref):
    e = e_ref[...]
    q = jnp.dot(e, q_ref[...], preferred_element_type=jnp.float32)
    k = jnp.dot(e, k_ref[...], preferred_element_type=jnp.float32)
    v = jnp.dot(e, v_ref[...], preferred_element_type=jnp.float32)
    att = jnp.sum(q * k, axis=1, keepdims=True)
    att = jnp.clip(att, -10.0, 10.0)
    ex = jnp.exp(att)
    an = ex / (ex + 1e-8)
    res_ref[...] = an * v
    att_ref[...] = an


def _dense(embeds, qT, kT, vT):
    n, d = embeds.shape
    rb = 1000
    grid = n // rb
    return pl.pallas_call(
        _dense_body,
        grid=(grid,),
        in_specs=[
            pl.BlockSpec((rb, d), lambda i: (i, 0)),
            pl.BlockSpec((d, d), lambda i: (0, 0)),
            pl.BlockSpec((d, d), lambda i: (0, 0)),
            pl.BlockSpec((d, d), lambda i: (0, 0)),
        ],
        out_specs=[
            pl.BlockSpec((rb, d), lambda i: (i, 0)),
            pl.BlockSpec((rb, 1), lambda i: (i, 0)),
        ],
        out_shape=[
            jax.ShapeDtypeStruct((n, d), jnp.float32),
            jax.ShapeDtypeStruct((n, 1), jnp.float32),
        ],
    )(embeds, qT, kT, vT)


NB = 4


def _spmm_body(n, nchunks, res_hbm, row_hbm, col_hbm, val_hbm, zero_hbm,
               out_hbm, idx_v, val_v, rows_b, acc_sh, isem, zsem, gsem, ssem):
    c = lax.axis_index("c")
    s = lax.axis_index("s")
    wid = s * NC + c
    d = res_hbm.shape[1]
    ndv = d // L
    ept = nchunks * CHUNK
    base_e = wid * ept

    rpt = (n // NS) // 8 * 8
    tail = n - NS * rpt
    pltpu.async_copy(zero_hbm.at[pl.ds(s * rpt, rpt)],
                     acc_sh.at[pl.ds(s * rpt, rpt)], zsem)

    @pl.when(s == NS - 1)
    def _():
        if tail > 0:
            pltpu.sync_copy(zero_hbm.at[pl.ds(NS * rpt, tail)],
                            acc_sh.at[pl.ds(NS * rpt, tail)])

    def idx_start(j, sl):
        off = base_e + j * CHUNK
        pltpu.async_copy(row_hbm.at[pl.ds(off, CHUNK)], idx_v.at[sl, 0],
                         isem.at[sl])
        pltpu.async_copy(col_hbm.at[pl.ds(off, CHUNK)], idx_v.at[sl, 1],
                         isem.at[sl])
        pltpu.async_copy(val_hbm.at[pl.ds(off, CHUNK)], val_v.at[sl],
                         isem.at[sl])

    def idx_wait(j, sl):
        off = base_e + j * CHUNK
        pltpu.make_async_copy(row_hbm.at[pl.ds(off, CHUNK)], idx_v.at[sl, 0],
                              isem.at[sl]).wait()
        pltpu.make_async_copy(col_hbm.at[pl.ds(off, CHUNK)], idx_v.at[sl, 1],
                              isem.at[sl]).wait()
        pltpu.make_async_copy(val_hbm.at[pl.ds(off, CHUNK)], val_v.at[sl],
                              isem.at[sl]).wait()

    def gather_start(b):
        pltpu.async_copy(res_hbm.at[idx_v.at[b, 1]], rows_b.at[b],
                         gsem.at[b])

    def gather_wait(b):
        pltpu.make_async_copy(res_hbm.at[idx_v.at[b, 1]], rows_b.at[b],
                              gsem.at[b]).wait()

    def scatter_start(b):
        pltpu.async_copy(rows_b.at[b], acc_sh.at[idx_v.at[b, 0]],
                         ssem.at[b], add=True)

    def scatter_wait(b):
        pltpu.make_async_copy(rows_b.at[b], acc_sh.at[idx_v.at[b, 0]],
                              ssem.at[b]).wait()

    idx_start(0, 0)
    idx_start(1, 1)
    pltpu.make_async_copy(zero_hbm.at[pl.ds(s * rpt, rpt)],
                          acc_sh.at[pl.ds(s * rpt, rpt)], zsem).wait()
    plsc.subcore_barrier()
    idx_wait(0, 0)
    gather_start(0)

    def outer(g, _):
        for k in range(NB):
            j = g * NB + k
            k1 = (k + 1) % NB
            k2 = (k + 2) % NB

            @pl.when(j >= 2)
            def _():
                scatter_wait(k2)

            @pl.when(j + 1 < nchunks)
            def _():
                idx_wait(j + 1, k1)
                gather_start(k1)

            @pl.when(j + 2 < nchunks)
            def _():
                idx_start(j + 2, k2)

            gather_wait(k)

            kk = jnp.full((L,), k, jnp.int32)

            @plsc.parallel_loop(0, CHUNK, unroll=2)
            def _(e):
                ee = jnp.broadcast_to(e, (L,)).astype(jnp.int32)
                vs = plsc.load_gather(val_v, [kk, ee])
                rb = rows_b.at[k]
                for grp in range(ndv):
                    sl = pl.ds(grp * L, L)
                    rb[e, sl] = rb[e, sl] * vs

            scatter_start(k)
        return _

    lax.fori_loop(0, nchunks // NB, outer, None)
    scatter_wait((nchunks - 2) % NB)
    scatter_wait((nchunks - 1) % NB)
    plsc.subcore_barrier()

    pltpu.sync_copy(acc_sh.at[pl.ds(s * rpt, rpt)],
                    out_hbm.at[c, pl.ds(s * rpt, rpt)])

    @pl.when(s == NS - 1)
    def _():
        if tail > 0:
            pltpu.sync_copy(acc_sh.at[pl.ds(NS * rpt, tail)],
                            out_hbm.at[c, pl.ds(NS * rpt, tail)])


def _spmm(res, rows, cols, vals, zeros):
    n, d = res.shape
    epad = rows.shape[0]
    nchunks = epad // (NW * CHUNK)
    mesh = plsc.VectorSubcoreMesh(core_axis_name="c", subcore_axis_name="s")
    kern = pl.kernel(
        functools.partial(_spmm_body, n, nchunks),
        out_type=jax.ShapeDtypeStruct((NC, n, d), jnp.float32),
        mesh=mesh,
        scratch_types=[
            pltpu.VMEM((NB, 2, CHUNK), jnp.int32),
            pltpu.VMEM((NB, CHUNK), jnp.float32),
            pltpu.VMEM((NB, CHUNK, d), jnp.float32),
            pltpu.VMEM_SHARED((n, d), jnp.float32),
            pltpu.SemaphoreType.DMA((NB,)),
            pltpu.SemaphoreType.DMA,
            pltpu.SemaphoreType.DMA((NB,)),
            pltpu.SemaphoreType.DMA((NB,)),
        ],
        compiler_params=pltpu.CompilerParams(needs_layout_passes=False),
    )
    return kern(res, rows, cols, vals, zeros)


def _combine_body(p_ref, o_ref):
    o_ref[...] = p_ref[0] + p_ref[1]


def _combine(partials):
    _, n, d = partials.shape
    rb = 1000
    return pl.pallas_call(
        _combine_body,
        grid=(n // rb,),
        in_specs=[pl.BlockSpec((2, rb, d), lambda i: (0, i, 0))],
        out_specs=pl.BlockSpec((rb, d), lambda i: (i, 0)),
        out_shape=jax.ShapeDtypeStruct((n, d), jnp.float32),
    )(partials)


def kernel(adj_indices, adj_values, embeds, qTrans, kTrans, vTrans):
    n, d = embeds.shape
    e = adj_values.shape[0]

    res, att_norm = _dense(embeds, qTrans, kTrans, vTrans)

    step = NW * CHUNK * NB
    epad = ((e + step - 1) // step) * step
    rows = jnp.zeros((epad,), jnp.int32).at[:e].set(
        adj_indices[0].astype(jnp.int32))
    cols = jnp.zeros((epad,), jnp.int32).at[:e].set(
        adj_indices[1].astype(jnp.int32))
    vals = jnp.zeros((epad,), jnp.float32).at[:e].set(adj_values)
    zeros = jnp.zeros((n, d), jnp.float32)

    partials = _spmm(res, rows, cols, vals, zeros)
    out = _combine(partials)
    return (out, att_norm)


# speedup vs baseline: 3.5419x; 3.5419x over previous
import functools

import jax
import jax.numpy as jnp
from jax import lax
from jax.experimental import pallas as pl
from jax.experimental.pallas import tpu as pltpu
from jax.experimental.pallas import tpu_sc as plsc

NC = 2
NS = 16
NW = NC * NS
L = 16

CHUNK = 128
NB = 3


def _dense_body(e_ref, q_ref, k_ref, v_ref, res_ref, att_ref):
    e = e_ref[...]
    q = jnp.dot(e, q_ref[...], preferred_element_type=jnp.float32)
    k = jnp.dot(e, k_ref[...], preferred_element_type=jnp.float32)
    v = jnp.dot(e, v_ref[...], preferred_element_type=jnp.float32)
    att = jnp.sum(q * k, axis=1, keepdims=True)
    att = jnp.clip(att, -10.0, 10.0)
    ex = jnp.exp(att)
    an = ex / (ex + 1e-8)
    res_ref[...] = an * v
    att_ref[...] = an


def _dense(embeds, qT, kT, vT):
    n, d = embeds.shape
    rb = 1000
    grid = n // rb
    return pl.pallas_call(
        _dense_body,
        grid=(grid,),
        in_specs=[
            pl.BlockSpec((rb, d), lambda i: (i, 0)),
            pl.BlockSpec((d, d), lambda i: (0, 0)),
            pl.BlockSpec((d, d), lambda i: (0, 0)),
            pl.BlockSpec((d, d), lambda i: (0, 0)),
        ],
        out_specs=[
            pl.BlockSpec((rb, d), lambda i: (i, 0)),
            pl.BlockSpec((rb, 1), lambda i: (i, 0)),
        ],
        out_shape=[
            jax.ShapeDtypeStruct((n, d), jnp.float32),
            jax.ShapeDtypeStruct((n, 1), jnp.float32),
        ],
    )(embeds, qT, kT, vT)


def _spmm_body(n, ept, res_hbm, row_hbm, col_hbm, val_hbm, zero_hbm,
               out_hbm, idx_v, val_v, tidx_v, tval_v, rows_b, acc_sh,
               isem, zsem, gsem, ssem):
    c = lax.axis_index("c")
    s = lax.axis_index("s")
    wid = s * NC + c
    d = res_hbm.shape[1]
    ndv = d // L
    nchunks = ept // CHUNK
    tail_e = ept - nchunks * CHUNK
    base_e = wid * ept

    rpt = (n // NS) // 8 * 8
    rtail = n - NS * rpt
    pltpu.async_copy(zero_hbm.at[pl.ds(s * rpt, rpt)],
                     acc_sh.at[pl.ds(s * rpt, rpt)], zsem)

    @pl.when(s == NS - 1)
    def _():
        if rtail > 0:
            pltpu.sync_copy(zero_hbm.at[pl.ds(NS * rpt, rtail)],
                            acc_sh.at[pl.ds(NS * rpt, rtail)])

    def idx_start(j, sl):
        off = base_e + j * CHUNK
        pltpu.async_copy(row_hbm.at[pl.ds(off, CHUNK)], idx_v.at[sl, 0],
                         isem.at[sl])
        pltpu.async_copy(col_hbm.at[pl.ds(off, CHUNK)], idx_v.at[sl, 1],
                         isem.at[sl])
        pltpu.async_copy(val_hbm.at[pl.ds(off, CHUNK)], val_v.at[sl],
                         isem.at[sl])

    def idx_wait(j, sl):
        off = base_e + j * CHUNK
        pltpu.make_async_copy(row_hbm.at[pl.ds(off, CHUNK)], idx_v.at[sl, 0],
                              isem.at[sl]).wait()
        pltpu.make_async_copy(col_hbm.at[pl.ds(off, CHUNK)], idx_v.at[sl, 1],
                              isem.at[sl]).wait()
        pltpu.make_async_copy(val_hbm.at[pl.ds(off, CHUNK)], val_v.at[sl],
                              isem.at[sl]).wait()

    def gather_start(b):
        pltpu.async_copy(res_hbm.at[idx_v.at[b, 1]], rows_b.at[b],
                         gsem.at[b])

    def gather_wait(b):
        pltpu.make_async_copy(res_hbm.at[idx_v.at[b, 1]], rows_b.at[b],
                              gsem.at[b]).wait()

    def scatter_start(b):
        pltpu.async_copy(rows_b.at[b], acc_sh.at[idx_v.at[b, 0]],
                         ssem.at[b], add=True)

    def scatter_wait(b):
        pltpu.make_async_copy(rows_b.at[b], acc_sh.at[idx_v.at[b, 0]],
                              ssem.at[b]).wait()

    idx_start(0, 0)
    idx_start(1, 1)
    pltpu.make_async_copy(zero_hbm.at[pl.ds(s * rpt, rpt)],
                          acc_sh.at[pl.ds(s * rpt, rpt)], zsem).wait()
    plsc.subcore_barrier()
    idx_wait(0, 0)
    gather_start(0)

    def outer(g, _):
        for k in range(NB):
            j = g * NB + k
            k1 = (k + 1) % NB
            k2 = (k + 2) % NB

            @pl.when(j >= 1)
            def _():
                scatter_wait(k2)

            @pl.when(j + 1 < nchunks)
            def _():
                idx_wait(j + 1, k1)
                gather_start(k1)

            @pl.when(j + 2 < nchunks)
            def _():
                idx_start(j + 2, k2)

            gather_wait(k)

            kk = jnp.full((L,), k, jnp.int32)

            @plsc.parallel_loop(0, CHUNK, unroll=2)
            def _(e):
                ee = jnp.broadcast_to(e, (L,)).astype(jnp.int32)
                vs = plsc.load_gather(val_v, [kk, ee])
                rb = rows_b.at[k]
                for grp in range(ndv):
                    sl = pl.ds(grp * L, L)
                    rb[e, sl] = rb[e, sl] * vs

            scatter_start(k)
        return _

    lax.fori_loop(0, nchunks // NB, outer, None)
    scatter_wait((nchunks - 1) % NB)

    if tail_e > 0:
        toff = base_e + nchunks * CHUNK
        pltpu.sync_copy(row_hbm.at[pl.ds(toff, tail_e)], tidx_v.at[0])
        pltpu.sync_copy(col_hbm.at[pl.ds(toff, tail_e)], tidx_v.at[1])
        pltpu.sync_copy(val_hbm.at[pl.ds(toff, tail_e)], tval_v)
        trows = rows_b.at[0, pl.ds(0, tail_e)]
        pltpu.async_copy(res_hbm.at[tidx_v.at[1]], trows, gsem.at[0]).wait()
        rb0 = rows_b.at[0]

        @plsc.parallel_loop(0, tail_e, unroll=2)
        def _(e):
            ee = jnp.broadcast_to(e, (L,)).astype(jnp.int32)
            vs = plsc.load_gather(tval_v, [ee])
            for grp in range(ndv):
                sl = pl.ds(grp * L, L)
                rb0[e, sl] = rb0[e, sl] * vs

        pltpu.async_copy(trows, acc_sh.at[tidx_v.at[0]], ssem.at[0],
                         add=True).wait()

    plsc.subcore_barrier()

    pltpu.sync_copy(acc_sh.at[pl.ds(s * rpt, rpt)],
                    out_hbm.at[c, pl.ds(s * rpt, rpt)])

    @pl.when(s == NS - 1)
    def _():
        if rtail > 0:
            pltpu.sync_copy(acc_sh.at[pl.ds(NS * rpt, rtail)],
                            out_hbm.at[c, pl.ds(NS * rpt, rtail)])


def _spmm(res, rows, cols, vals, zeros):
    n, d = res.shape
    e = rows.shape[0]
    ept = e // NW
    tail_e = ept - (ept // CHUNK) * CHUNK
    tail_e = max(tail_e, L)
    mesh = plsc.VectorSubcoreMesh(core_axis_name="c", subcore_axis_name="s")
    kern = pl.kernel(
        functools.partial(_spmm_body, n, ept),
        out_type=jax.ShapeDtypeStruct((NC, n, d), jnp.float32),
        mesh=mesh,
        scratch_types=[
            pltpu.VMEM((NB, 2, CHUNK), jnp.int32),
            pltpu.VMEM((NB, CHUNK), jnp.float32),
            pltpu.VMEM((2, tail_e), jnp.int32),
            pltpu.VMEM((tail_e,), jnp.float32),
            pltpu.VMEM((NB, CHUNK, d), jnp.float32),
            pltpu.VMEM_SHARED((n, d), jnp.float32),
            pltpu.SemaphoreType.DMA((NB,)),
            pltpu.SemaphoreType.DMA,
            pltpu.SemaphoreType.DMA((NB,)),
            pltpu.SemaphoreType.DMA((NB,)),
        ],
        compiler_params=pltpu.CompilerParams(needs_layout_passes=False),
    )
    return kern(res, rows, cols, vals, zeros)


def _combine_body(p_ref, o_ref):
    o_ref[...] = p_ref[0] + p_ref[1]


def _combine(partials):
    _, n, d = partials.shape
    rb = 1000
    return pl.pallas_call(
        _combine_body,
        grid=(n // rb,),
        in_specs=[pl.BlockSpec((2, rb, d), lambda i: (0, i, 0))],
        out_specs=pl.BlockSpec((rb, d), lambda i: (i, 0)),
        out_shape=jax.ShapeDtypeStruct((n, d), jnp.float32),
    )(partials)


def kernel(adj_indices, adj_values, embeds, qTrans, kTrans, vTrans):
    n, d = embeds.shape

    res, att_norm = _dense(embeds, qTrans, kTrans, vTrans)

    rows = adj_indices[0].astype(jnp.int32)
    cols = adj_indices[1].astype(jnp.int32)
    vals = adj_values
    zeros = jnp.zeros((n, d), jnp.float32)

    partials = _spmm(res, rows, cols, vals, zeros)
    out = _combine(partials)
    return (out, att_norm)


# speedup vs baseline: 3.6134x; 1.0202x over previous
import functools

import jax
import jax.numpy as jnp
from jax import lax
from jax.experimental import pallas as pl
from jax.experimental.pallas import tpu as pltpu
from jax.experimental.pallas import tpu_sc as plsc

NC = 2
NS = 16
NW = NC * NS
L = 16

CHUNK = 128
NB = 3


def _dense_body(e_ref, q_ref, k_ref, v_ref, res_ref, att_ref):
    e = e_ref[...]
    q = jnp.dot(e, q_ref[...], preferred_element_type=jnp.float32)
    k = jnp.dot(e, k_ref[...], preferred_element_type=jnp.float32)
    v = jnp.dot(e, v_ref[...], preferred_element_type=jnp.float32)
    att = jnp.sum(q * k, axis=1, keepdims=True)
    att = jnp.clip(att, -10.0, 10.0)
    ex = jnp.exp(att)
    an = ex / (ex + 1e-8)
    res_ref[...] = an * v
    att_ref[...] = an


def _dense(embeds, qT, kT, vT):
    n, d = embeds.shape
    rb = 1000
    grid = n // rb
    return pl.pallas_call(
        _dense_body,
        grid=(grid,),
        in_specs=[
            pl.BlockSpec((rb, d), lambda i: (i, 0)),
            pl.BlockSpec((d, d), lambda i: (0, 0)),
            pl.BlockSpec((d, d), lambda i: (0, 0)),
            pl.BlockSpec((d, d), lambda i: (0, 0)),
        ],
        out_specs=[
            pl.BlockSpec((rb, d), lambda i: (i, 0)),
            pl.BlockSpec((rb, 1), lambda i: (i, 0)),
        ],
        out_shape=[
            jax.ShapeDtypeStruct((n, d), jnp.float32),
            jax.ShapeDtypeStruct((n, 1), jnp.float32),
        ],
    )(embeds, qT, kT, vT)


def _spmm_body(n, ept, res_hbm, row_hbm, col_hbm, val_hbm, zero_hbm,
               out_hbm, idx_v, val_v, tidx_v, tval_v, rows_b, acc_sh,
               isem, zsem, gsem, ssem):
    c = lax.axis_index("c")
    s = lax.axis_index("s")
    wid = s * NC + c
    d = res_hbm.shape[1]
    ndv = d // L
    nchunks = ept // CHUNK
    tail_e = ept - nchunks * CHUNK
    base_e = wid * ept

    rpt = (n // NS) // 8 * 8
    rtail = n - NS * rpt
    pltpu.async_copy(zero_hbm.at[pl.ds(s * rpt, rpt)],
                     acc_sh.at[pl.ds(s * rpt, rpt)], zsem)

    @pl.when(s == NS - 1)
    def _():
        if rtail > 0:
            pltpu.sync_copy(zero_hbm.at[pl.ds(NS * rpt, rtail)],
                            acc_sh.at[pl.ds(NS * rpt, rtail)])

    def idx_start(j, sl):
        off = base_e + j * CHUNK
        pltpu.async_copy(row_hbm.at[pl.ds(off, CHUNK)], idx_v.at[sl, 0],
                         isem.at[sl])
        pltpu.async_copy(col_hbm.at[pl.ds(off, CHUNK)], idx_v.at[sl, 1],
                         isem.at[sl])
        pltpu.async_copy(val_hbm.at[pl.ds(off, CHUNK)], val_v.at[sl],
                         isem.at[sl])

    def idx_wait(j, sl):
        off = base_e + j * CHUNK
        pltpu.make_async_copy(row_hbm.at[pl.ds(off, CHUNK)], idx_v.at[sl, 0],
                              isem.at[sl]).wait()
        pltpu.make_async_copy(col_hbm.at[pl.ds(off, CHUNK)], idx_v.at[sl, 1],
                              isem.at[sl]).wait()
        pltpu.make_async_copy(val_hbm.at[pl.ds(off, CHUNK)], val_v.at[sl],
                              isem.at[sl]).wait()

    def gather_start(b):
        pltpu.async_copy(res_hbm.at[idx_v.at[b, 1]], rows_b.at[b],
                         gsem.at[b])

    def gather_wait(b):
        pltpu.make_async_copy(res_hbm.at[idx_v.at[b, 1]], rows_b.at[b],
                              gsem.at[b]).wait()

    def scatter_start(b):
        pltpu.async_copy(rows_b.at[b], acc_sh.at[idx_v.at[b, 0]],
                         ssem.at[b], add=True)

    def scatter_wait(b):
        pltpu.make_async_copy(rows_b.at[b], acc_sh.at[idx_v.at[b, 0]],
                              ssem.at[b]).wait()

    idx_start(0, 0)
    idx_start(1, 1)
    pltpu.make_async_copy(zero_hbm.at[pl.ds(s * rpt, rpt)],
                          acc_sh.at[pl.ds(s * rpt, rpt)], zsem).wait()
    plsc.subcore_barrier()
    idx_wait(0, 0)
    gather_start(0)

    def outer(g, _):
        for k in range(NB):
            j = g * NB + k
            k1 = (k + 1) % NB
            k2 = (k + 2) % NB

            @pl.when(j + 1 < nchunks)
            def _():
                idx_wait(j + 1, k1)
                gather_start(k1)

            @pl.when(j >= 1)
            def _():
                scatter_wait(k2)

            @pl.when(j + 2 < nchunks)
            def _():
                idx_start(j + 2, k2)

            gather_wait(k)

            kk = jnp.full((L,), k, jnp.int32)

            @plsc.parallel_loop(0, CHUNK, unroll=2)
            def _(e):
                ee = jnp.broadcast_to(e, (L,)).astype(jnp.int32)
                vs = plsc.load_gather(val_v, [kk, ee])
                rb = rows_b.at[k]
                for grp in range(ndv):
                    sl = pl.ds(grp * L, L)
                    rb[e, sl] = rb[e, sl] * vs

            scatter_start(k)
        return _

    lax.fori_loop(0, nchunks // NB, outer, None)
    scatter_wait((nchunks - 1) % NB)

    if tail_e > 0:
        toff = base_e + nchunks * CHUNK
        pltpu.sync_copy(row_hbm.at[pl.ds(toff, tail_e)], tidx_v.at[0])
        pltpu.sync_copy(col_hbm.at[pl.ds(toff, tail_e)], tidx_v.at[1])
        pltpu.sync_copy(val_hbm.at[pl.ds(toff, tail_e)], tval_v)
        trows = rows_b.at[0, pl.ds(0, tail_e)]
        pltpu.async_copy(res_hbm.at[tidx_v.at[1]], trows, gsem.at[0]).wait()
        rb0 = rows_b.at[0]

        @plsc.parallel_loop(0, tail_e, unroll=2)
        def _(e):
            ee = jnp.broadcast_to(e, (L,)).astype(jnp.int32)
            vs = plsc.load_gather(tval_v, [ee])
            for grp in range(ndv):
                sl = pl.ds(grp * L, L)
                rb0[e, sl] = rb0[e, sl] * vs

        pltpu.async_copy(trows, acc_sh.at[tidx_v.at[0]], ssem.at[0],
                         add=True).wait()

    plsc.subcore_barrier()

    pltpu.sync_copy(acc_sh.at[pl.ds(s * rpt, rpt)],
                    out_hbm.at[c, pl.ds(s * rpt, rpt)])

    @pl.when(s == NS - 1)
    def _():
        if rtail > 0:
            pltpu.sync_copy(acc_sh.at[pl.ds(NS * rpt, rtail)],
                            out_hbm.at[c, pl.ds(NS * rpt, rtail)])


def _spmm(res, rows, cols, vals, zeros):
    n, d = res.shape
    e = rows.shape[0]
    ept = e // NW
    tail_e = ept - (ept // CHUNK) * CHUNK
    tail_e = max(tail_e, L)
    mesh = plsc.VectorSubcoreMesh(core_axis_name="c", subcore_axis_name="s")
    kern = pl.kernel(
        functools.partial(_spmm_body, n, ept),
        out_type=jax.ShapeDtypeStruct((NC, n, d), jnp.float32),
        mesh=mesh,
        scratch_types=[
            pltpu.VMEM((NB, 2, CHUNK), jnp.int32),
            pltpu.VMEM((NB, CHUNK), jnp.float32),
            pltpu.VMEM((2, tail_e), jnp.int32),
            pltpu.VMEM((tail_e,), jnp.float32),
            pltpu.VMEM((NB, CHUNK, d), jnp.float32),
            pltpu.VMEM_SHARED((n, d), jnp.float32),
            pltpu.SemaphoreType.DMA((NB,)),
            pltpu.SemaphoreType.DMA,
            pltpu.SemaphoreType.DMA((NB,)),
            pltpu.SemaphoreType.DMA((NB,)),
        ],
        compiler_params=pltpu.CompilerParams(needs_layout_passes=False),
    )
    return kern(res, rows, cols, vals, zeros)


def _combine_body(p_ref, o_ref):
    o_ref[...] = p_ref[0] + p_ref[1]


def _combine(partials):
    _, n, d = partials.shape
    rb = 1000
    return pl.pallas_call(
        _combine_body,
        grid=(n // rb,),
        in_specs=[pl.BlockSpec((2, rb, d), lambda i: (0, i, 0))],
        out_specs=pl.BlockSpec((rb, d), lambda i: (i, 0)),
        out_shape=jax.ShapeDtypeStruct((n, d), jnp.float32),
    )(partials)


def kernel(adj_indices, adj_values, embeds, qTrans, kTrans, vTrans):
    n, d = embeds.shape

    res, att_norm = _dense(embeds, qTrans, kTrans, vTrans)

    rows = adj_indices[0].astype(jnp.int32)
    cols = adj_indices[1].astype(jnp.int32)
    zeros = jnp.zeros((n, d), jnp.float32)

    partials = _spmm(res, rows, cols, adj_values, zeros)
    out = _combine(partials)
    return (out, att_norm)


# speedup vs baseline: 3.7494x; 1.0376x over previous
import functools

import jax
import jax.numpy as jnp
from jax import lax
from jax.experimental import pallas as pl
from jax.experimental.pallas import tpu as pltpu
from jax.experimental.pallas import tpu_sc as plsc

NC = 2
NS = 16
NW = NC * NS
L = 16

CHUNK = 128
NB = 3


def _dense_body(e_ref, q_ref, k_ref, v_ref, res_ref, att_ref, zero_ref):
    e = e_ref[...]
    q = jnp.dot(e, q_ref[...], preferred_element_type=jnp.float32)
    k = jnp.dot(e, k_ref[...], preferred_element_type=jnp.float32)
    v = jnp.dot(e, v_ref[...], preferred_element_type=jnp.float32)
    att = jnp.sum(q * k, axis=1, keepdims=True)
    att = jnp.clip(att, -10.0, 10.0)
    ex = jnp.exp(att)
    an = ex / (ex + 1e-8)
    res_ref[...] = an * v
    att_ref[...] = an
    zero_ref[...] = jnp.zeros_like(zero_ref)


def _dense(embeds, qT, kT, vT):
    n, d = embeds.shape
    rb = 2000
    grid = n // rb
    return pl.pallas_call(
        _dense_body,
        grid=(grid,),
        in_specs=[
            pl.BlockSpec((rb, d), lambda i: (i, 0)),
            pl.BlockSpec((d, d), lambda i: (0, 0)),
            pl.BlockSpec((d, d), lambda i: (0, 0)),
            pl.BlockSpec((d, d), lambda i: (0, 0)),
        ],
        out_specs=[
            pl.BlockSpec((rb, d), lambda i: (i, 0)),
            pl.BlockSpec((rb, 1), lambda i: (i, 0)),
            pl.BlockSpec((rb, d), lambda i: (i, 0)),
        ],
        out_shape=[
            jax.ShapeDtypeStruct((n, d), jnp.float32),
            jax.ShapeDtypeStruct((n, 1), jnp.float32),
            jax.ShapeDtypeStruct((n, d), jnp.float32),
        ],
    )(embeds, qT, kT, vT)


def _spmm_body(n, ept, res_hbm, row_hbm, col_hbm, val_hbm, zero_hbm,
               out_hbm, idx_v, val_v, tidx_v, tval_v, rows_b, acc_sh,
               isem, zsem, gsem, ssem):
    c = lax.axis_index("c")
    s = lax.axis_index("s")
    wid = s * NC + c
    d = res_hbm.shape[1]
    ndv = d // L
    nchunks = ept // CHUNK
    tail_e = ept - nchunks * CHUNK
    base_e = wid * ept

    rpt = (n // NS) // 8 * 8
    rtail = n - NS * rpt
    pltpu.async_copy(zero_hbm.at[pl.ds(s * rpt, rpt)],
                     acc_sh.at[pl.ds(s * rpt, rpt)], zsem)

    @pl.when(s == NS - 1)
    def _():
        if rtail > 0:
            pltpu.sync_copy(zero_hbm.at[pl.ds(NS * rpt, rtail)],
                            acc_sh.at[pl.ds(NS * rpt, rtail)])

    def idx_start(j, sl):
        off = base_e + j * CHUNK
        pltpu.async_copy(row_hbm.at[pl.ds(off, CHUNK)], idx_v.at[sl, 0],
                         isem.at[sl])
        pltpu.async_copy(col_hbm.at[pl.ds(off, CHUNK)], idx_v.at[sl, 1],
                         isem.at[sl])
        pltpu.async_copy(val_hbm.at[pl.ds(off, CHUNK)], val_v.at[sl],
                         isem.at[sl])

    def idx_wait(j, sl):
        off = base_e + j * CHUNK
        pltpu.make_async_copy(row_hbm.at[pl.ds(off, CHUNK)], idx_v.at[sl, 0],
                              isem.at[sl]).wait()
        pltpu.make_async_copy(col_hbm.at[pl.ds(off, CHUNK)], idx_v.at[sl, 1],
                              isem.at[sl]).wait()
        pltpu.make_async_copy(val_hbm.at[pl.ds(off, CHUNK)], val_v.at[sl],
                              isem.at[sl]).wait()

    def gather_start(b):
        pltpu.async_copy(res_hbm.at[idx_v.at[b, 1]], rows_b.at[b],
                         gsem.at[b])

    def gather_wait(b):
        pltpu.make_async_copy(res_hbm.at[idx_v.at[b, 1]], rows_b.at[b],
                              gsem.at[b]).wait()

    def scatter_start(b):
        pltpu.async_copy(rows_b.at[b], acc_sh.at[idx_v.at[b, 0]],
                         ssem.at[b], add=True)

    def scatter_wait(b):
        pltpu.make_async_copy(rows_b.at[b], acc_sh.at[idx_v.at[b, 0]],
                              ssem.at[b]).wait()

    idx_start(0, 0)
    idx_start(1, 1)
    pltpu.make_async_copy(zero_hbm.at[pl.ds(s * rpt, rpt)],
                          acc_sh.at[pl.ds(s * rpt, rpt)], zsem).wait()
    plsc.subcore_barrier()
    idx_wait(0, 0)
    gather_start(0)

    def outer(g, _):
        for k in range(NB):
            j = g * NB + k
            k1 = (k + 1) % NB
            k2 = (k + 2) % NB

            @pl.when(j + 1 < nchunks)
            def _():
                idx_wait(j + 1, k1)
                gather_start(k1)

            @pl.when(j >= 1)
            def _():
                scatter_wait(k2)

            @pl.when(j + 2 < nchunks)
            def _():
                idx_start(j + 2, k2)

            gather_wait(k)

            kk = jnp.full((L,), k, jnp.int32)

            @plsc.parallel_loop(0, CHUNK, unroll=2)
            def _(e):
                ee = jnp.broadcast_to(e, (L,)).astype(jnp.int32)
                vs = plsc.load_gather(val_v, [kk, ee])
                rb = rows_b.at[k]
                for grp in range(ndv):
                    sl = pl.ds(grp * L, L)
                    rb[e, sl] = rb[e, sl] * vs

            scatter_start(k)
        return _

    lax.fori_loop(0, nchunks // NB, outer, None)
    scatter_wait((nchunks - 1) % NB)

    if tail_e > 0:
        toff = base_e + nchunks * CHUNK
        pltpu.sync_copy(row_hbm.at[pl.ds(toff, tail_e)], tidx_v.at[0])
        pltpu.sync_copy(col_hbm.at[pl.ds(toff, tail_e)], tidx_v.at[1])
        pltpu.sync_copy(val_hbm.at[pl.ds(toff, tail_e)], tval_v)
        trows = rows_b.at[0, pl.ds(0, tail_e)]
        pltpu.async_copy(res_hbm.at[tidx_v.at[1]], trows, gsem.at[0]).wait()
        rb0 = rows_b.at[0]

        @plsc.parallel_loop(0, tail_e, unroll=2)
        def _(e):
            ee = jnp.broadcast_to(e, (L,)).astype(jnp.int32)
            vs = plsc.load_gather(tval_v, [ee])
            for grp in range(ndv):
                sl = pl.ds(grp * L, L)
                rb0[e, sl] = rb0[e, sl] * vs

        pltpu.async_copy(trows, acc_sh.at[tidx_v.at[0]], ssem.at[0],
                         add=True).wait()

    plsc.subcore_barrier()

    pltpu.sync_copy(acc_sh.at[pl.ds(s * rpt, rpt)],
                    out_hbm.at[c, pl.ds(s * rpt, rpt)])

    @pl.when(s == NS - 1)
    def _():
        if rtail > 0:
            pltpu.sync_copy(acc_sh.at[pl.ds(NS * rpt, rtail)],
                            out_hbm.at[c, pl.ds(NS * rpt, rtail)])


def _spmm(res, rows, cols, vals, zeros):
    n, d = res.shape
    e = rows.shape[0]
    ept = e // NW
    tail_e = ept - (ept // CHUNK) * CHUNK
    tail_e = max(tail_e, L)
    mesh = plsc.VectorSubcoreMesh(core_axis_name="c", subcore_axis_name="s")
    kern = pl.kernel(
        functools.partial(_spmm_body, n, ept),
        out_type=jax.ShapeDtypeStruct((NC, n, d), jnp.float32),
        mesh=mesh,
        scratch_types=[
            pltpu.VMEM((NB, 2, CHUNK), jnp.int32),
            pltpu.VMEM((NB, CHUNK), jnp.float32),
            pltpu.VMEM((2, tail_e), jnp.int32),
            pltpu.VMEM((tail_e,), jnp.float32),
            pltpu.VMEM((NB, CHUNK, d), jnp.float32),
            pltpu.VMEM_SHARED((n, d), jnp.float32),
            pltpu.SemaphoreType.DMA((NB,)),
            pltpu.SemaphoreType.DMA,
            pltpu.SemaphoreType.DMA((NB,)),
            pltpu.SemaphoreType.DMA((NB,)),
        ],
        compiler_params=pltpu.CompilerParams(needs_layout_passes=False),
    )
    return kern(res, rows, cols, vals, zeros)


def _combine_body(p_ref, o_ref):
    o_ref[...] = p_ref[0] + p_ref[1]


def _combine(partials):
    _, n, d = partials.shape
    rb = 2000
    return pl.pallas_call(
        _combine_body,
        grid=(n // rb,),
        in_specs=[pl.BlockSpec((2, rb, d), lambda i: (0, i, 0))],
        out_specs=pl.BlockSpec((rb, d), lambda i: (i, 0)),
        out_shape=jax.ShapeDtypeStruct((n, d), jnp.float32),
    )(partials)


def kernel(adj_indices, adj_values, embeds, qTrans, kTrans, vTrans):
    n, d = embeds.shape

    res, att_norm, zeros = _dense(embeds, qTrans, kTrans, vTrans)

    rows = adj_indices[0].astype(jnp.int32)
    cols = adj_indices[1].astype(jnp.int32)

    partials = _spmm(res, rows, cols, adj_values, zeros)
    out = _combine(partials)
    return (out, att_norm)


# speedup vs baseline: 3.7653x; 1.0042x over previous
import functools

import jax
import jax.numpy as jnp
from jax import lax
from jax.experimental import pallas as pl
from jax.experimental.pallas import tpu as pltpu
from jax.experimental.pallas import tpu_sc as plsc

NC = 2
NS = 16
NW = NC * NS
L = 16

CHUNK = 128
NB = 3


def _dense_body(e_ref, q_ref, k_ref, v_ref, res_ref, att_ref, zero_ref):
    e = e_ref[...]
    q = jnp.dot(e, q_ref[...], preferred_element_type=jnp.float32)
    k = jnp.dot(e, k_ref[...], preferred_element_type=jnp.float32)
    v = jnp.dot(e, v_ref[...], preferred_element_type=jnp.float32)
    att = jnp.sum(q * k, axis=1, keepdims=True)
    att = jnp.clip(att, -10.0, 10.0)
    ex = jnp.exp(att)
    an = ex / (ex + 1e-8)
    res_ref[...] = an * v
    att_ref[...] = an
    zero_ref[...] = jnp.zeros_like(zero_ref)


def _dense(embeds, qT, kT, vT):
    n, d = embeds.shape
    rb = 2000
    grid = n // rb
    return pl.pallas_call(
        _dense_body,
        grid=(grid,),
        in_specs=[
            pl.BlockSpec((rb, d), lambda i: (i, 0)),
            pl.BlockSpec((d, d), lambda i: (0, 0)),
            pl.BlockSpec((d, d), lambda i: (0, 0)),
            pl.BlockSpec((d, d), lambda i: (0, 0)),
        ],
        out_specs=[
            pl.BlockSpec((rb, d), lambda i: (i, 0)),
            pl.BlockSpec((rb, 1), lambda i: (i, 0)),
            pl.BlockSpec((rb, d), lambda i: (i, 0)),
        ],
        out_shape=[
            jax.ShapeDtypeStruct((n, d), jnp.float32),
            jax.ShapeDtypeStruct((n, 1), jnp.float32),
            jax.ShapeDtypeStruct((n, d), jnp.float32),
        ],
    )(embeds, qT, kT, vT)


def _spmm_body(n, ept, res_hbm, row_hbm, col_hbm, val_hbm, zero_hbm,
               out_hbm, idx_v, val_v, tidx_v, tval_v, rows_b, acc_sh,
               isem, zsem, gsem, ssem):
    c = lax.axis_index("c")
    s = lax.axis_index("s")
    wid = s * NC + c
    d = res_hbm.shape[1]
    ndv = d // L
    nchunks = ept // CHUNK
    tail_e = ept - nchunks * CHUNK
    base_e = wid * ept

    rpt = (n // NS) // 8 * 8
    rtail = n - NS * rpt
    pltpu.async_copy(zero_hbm.at[pl.ds(s * rpt, rpt)],
                     acc_sh.at[pl.ds(s * rpt, rpt)], zsem)

    @pl.when(s == NS - 1)
    def _():
        if rtail > 0:
            pltpu.sync_copy(zero_hbm.at[pl.ds(NS * rpt, rtail)],
                            acc_sh.at[pl.ds(NS * rpt, rtail)])

    def idx_start(j, sl):
        off = base_e + j * CHUNK
        pltpu.async_copy(row_hbm.at[pl.ds(off, CHUNK)], idx_v.at[sl, 0],
                         isem.at[sl])
        pltpu.async_copy(col_hbm.at[pl.ds(off, CHUNK)], idx_v.at[sl, 1],
                         isem.at[sl])
        pltpu.async_copy(val_hbm.at[pl.ds(off, CHUNK)], val_v.at[sl],
                         isem.at[sl])

    def idx_wait(j, sl):
        off = base_e + j * CHUNK
        pltpu.make_async_copy(row_hbm.at[pl.ds(off, CHUNK)], idx_v.at[sl, 0],
                              isem.at[sl]).wait()
        pltpu.make_async_copy(col_hbm.at[pl.ds(off, CHUNK)], idx_v.at[sl, 1],
                              isem.at[sl]).wait()
        pltpu.make_async_copy(val_hbm.at[pl.ds(off, CHUNK)], val_v.at[sl],
                              isem.at[sl]).wait()

    def gather_start(b):
        pltpu.async_copy(res_hbm.at[idx_v.at[b, 1]], rows_b.at[b],
                         gsem.at[b])

    def gather_wait(b):
        pltpu.make_async_copy(res_hbm.at[idx_v.at[b, 1]], rows_b.at[b],
                              gsem.at[b]).wait()

    def scatter_start(b):
        pltpu.async_copy(rows_b.at[b], acc_sh.at[idx_v.at[b, 0]],
                         ssem.at[b], add=True)

    def scatter_wait(b):
        pltpu.make_async_copy(rows_b.at[b], acc_sh.at[idx_v.at[b, 0]],
                              ssem.at[b]).wait()

    idx_start(0, 0)
    idx_start(1, 1)
    pltpu.make_async_copy(zero_hbm.at[pl.ds(s * rpt, rpt)],
                          acc_sh.at[pl.ds(s * rpt, rpt)], zsem).wait()
    plsc.subcore_barrier()
    idx_wait(0, 0)
    gather_start(0)

    def outer(g, _):
        for k in range(NB):
            j = g * NB + k
            k1 = (k + 1) % NB
            k2 = (k + 2) % NB

            @pl.when(j + 1 < nchunks)
            def _():
                idx_wait(j + 1, k1)
                gather_start(k1)

            @pl.when(j >= 1)
            def _():
                scatter_wait(k2)

            @pl.when(j + 2 < nchunks)
            def _():
                idx_start(j + 2, k2)

            gather_wait(k)

            kk = jnp.full((L,), k, jnp.int32)

            @plsc.parallel_loop(0, CHUNK, unroll=4,
                                carry=jnp.zeros((L,), jnp.int32))
            def _mul(e, ev):
                vs = plsc.load_gather(val_v, [kk, ev])
                rb = rows_b.at[k]
                for grp in range(ndv):
                    sl = pl.ds(grp * L, L)
                    rb[e, sl] = rb[e, sl] * vs
                return ev + 1
            del _mul

            scatter_start(k)
        return _

    lax.fori_loop(0, nchunks // NB, outer, None)
    scatter_wait((nchunks - 1) % NB)

    if tail_e > 0:
        toff = base_e + nchunks * CHUNK
        pltpu.sync_copy(row_hbm.at[pl.ds(toff, tail_e)], tidx_v.at[0])
        pltpu.sync_copy(col_hbm.at[pl.ds(toff, tail_e)], tidx_v.at[1])
        pltpu.sync_copy(val_hbm.at[pl.ds(toff, tail_e)], tval_v)
        trows = rows_b.at[0, pl.ds(0, tail_e)]
        pltpu.async_copy(res_hbm.at[tidx_v.at[1]], trows, gsem.at[0]).wait()
        rb0 = rows_b.at[0]

        @plsc.parallel_loop(0, tail_e, unroll=2)
        def _(e):
            ee = jnp.broadcast_to(e, (L,)).astype(jnp.int32)
            vs = plsc.load_gather(tval_v, [ee])
            for grp in range(ndv):
                sl = pl.ds(grp * L, L)
                rb0[e, sl] = rb0[e, sl] * vs

        pltpu.async_copy(trows, acc_sh.at[tidx_v.at[0]], ssem.at[0],
                         add=True).wait()

    plsc.subcore_barrier()

    pltpu.sync_copy(acc_sh.at[pl.ds(s * rpt, rpt)],
                    out_hbm.at[c, pl.ds(s * rpt, rpt)])

    @pl.when(s == NS - 1)
    def _():
        if rtail > 0:
            pltpu.sync_copy(acc_sh.at[pl.ds(NS * rpt, rtail)],
                            out_hbm.at[c, pl.ds(NS * rpt, rtail)])


def _spmm(res, rows, cols, vals, zeros):
    n, d = res.shape
    e = rows.shape[0]
    ept = e // NW
    tail_e = ept - (ept // CHUNK) * CHUNK
    tail_e = max(tail_e, L)
    mesh = plsc.VectorSubcoreMesh(core_axis_name="c", subcore_axis_name="s")
    kern = pl.kernel(
        functools.partial(_spmm_body, n, ept),
        out_type=jax.ShapeDtypeStruct((NC, n, d), jnp.float32),
        mesh=mesh,
        scratch_types=[
            pltpu.VMEM((NB, 2, CHUNK), jnp.int32),
            pltpu.VMEM((NB, CHUNK), jnp.float32),
            pltpu.VMEM((2, tail_e), jnp.int32),
            pltpu.VMEM((tail_e,), jnp.float32),
            pltpu.VMEM((NB, CHUNK, d), jnp.float32),
            pltpu.VMEM_SHARED((n, d), jnp.float32),
            pltpu.SemaphoreType.DMA((NB,)),
            pltpu.SemaphoreType.DMA,
            pltpu.SemaphoreType.DMA((NB,)),
            pltpu.SemaphoreType.DMA((NB,)),
        ],
        compiler_params=pltpu.CompilerParams(needs_layout_passes=False),
    )
    return kern(res, rows, cols, vals, zeros)


def _combine_body(p_ref, o_ref):
    o_ref[...] = p_ref[0] + p_ref[1]


def _combine(partials):
    _, n, d = partials.shape
    rb = 2000
    return pl.pallas_call(
        _combine_body,
        grid=(n // rb,),
        in_specs=[pl.BlockSpec((2, rb, d), lambda i: (0, i, 0))],
        out_specs=pl.BlockSpec((rb, d), lambda i: (i, 0)),
        out_shape=jax.ShapeDtypeStruct((n, d), jnp.float32),
    )(partials)


def kernel(adj_indices, adj_values, embeds, qTrans, kTrans, vTrans):
    n, d = embeds.shape

    res, att_norm, zeros = _dense(embeds, qTrans, kTrans, vTrans)

    rows = adj_indices[0].astype(jnp.int32)
    cols = adj_indices[1].astype(jnp.int32)

    partials = _spmm(res, rows, cols, adj_values, zeros)
    out = _combine(partials)
    return (out, att_norm)
